# baseline (speedup 1.0000x reference)
import functools
import jax
import jax.numpy as jnp
import numpy as np
from jax import lax
from jax.experimental import pallas as pl
from jax.experimental.pallas import tpu as pltpu
from jax.experimental.pallas import tpu_sc as plsc

N_PT = 512
RAD = 0.2
N_SMP = 32
EPSV = 1e-5
TS = 512


def _fps_body(xs_ref, ys_ref, zs_ref, far0_ref, ce_ref, cx_ref, cy_ref, cz_ref):
    B, N = xs_ref.shape
    xs, ys, zs = xs_ref[...], ys_ref[...], zs_ref[...]
    lane = jax.lax.broadcasted_iota(jnp.int32, (B, N), 1)
    slot = jax.lax.broadcasted_iota(jnp.int32, (B, N_PT), 1)

    def body(i, st):
        d, far, ce, cx, cy, cz = st
        mask = lane == far
        cxi = jnp.sum(jnp.where(mask, xs, 0.0), axis=1, keepdims=True)
        cyi = jnp.sum(jnp.where(mask, ys, 0.0), axis=1, keepdims=True)
        czi = jnp.sum(jnp.where(mask, zs, 0.0), axis=1, keepdims=True)
        dx = xs - cxi
        dy = ys - cyi
        dz = zs - czi
        dd = dx * dx + dy * dy + dz * dz
        d = jnp.minimum(d, dd)
        m = jnp.max(d, axis=1, keepdims=True)
        nfar = jnp.min(jnp.where(d == m, lane, N), axis=1, keepdims=True)
        at = slot == i
        ce = jnp.where(at, far, ce)
        cx = jnp.where(at, cxi, cx)
        cy = jnp.where(at, cyi, cy)
        cz = jnp.where(at, czi, cz)
        return (d, nfar, ce, cx, cy, cz)

    d0 = jnp.full((B, N), 1e10, jnp.float32)
    far0 = far0_ref[...][:, :1]
    z512i = jnp.zeros((B, N_PT), jnp.int32)
    z512f = jnp.zeros((B, N_PT), jnp.float32)
    _, _, ce, cx, cy, cz = jax.lax.fori_loop(
        0, N_PT, body, (d0, far0, z512i, z512f, z512f, z512f))
    ce_ref[...] = ce
    cx_ref[...] = cx
    cy_ref[...] = cy
    cz_ref[...] = cz


def _fps_pallas(xyz):
    B, N, _ = xyz.shape
    far0 = jax.random.randint(jax.random.key(42), (B,), 0, N).astype(jnp.int32)
    far0 = jnp.broadcast_to(far0[:, None], (B, 128))
    xT = xyz.transpose(2, 0, 1)
    ce, cx, cy, cz = pl.pallas_call(
        _fps_body,
        out_shape=[
            jax.ShapeDtypeStruct((B, N_PT), jnp.int32),
            jax.ShapeDtypeStruct((B, N_PT), jnp.float32),
            jax.ShapeDtypeStruct((B, N_PT), jnp.float32),
            jax.ShapeDtypeStruct((B, N_PT), jnp.float32),
        ],
    )(xT[0], xT[1], xT[2], far0)
    new_xyz = jnp.stack([cx, cy, cz], axis=-1)
    nxT = jnp.stack([cx, cy, cz], axis=0).reshape(3, B * N_PT)
    return ce, new_xyz, nxT


_GCH = 128


def _sc_gather(fidx, ptab, S):
    mesh = plsc.VectorSubcoreMesh(core_axis_name="c", subcore_axis_name="s")
    NC, NS = 2, 16
    NW = NC * NS
    rows_w = S // NW
    nch = rows_w // _GCH
    CW = ptab.shape[1]

    @functools.partial(
        pl.kernel,
        mesh=mesh,
        out_type=jax.ShapeDtypeStruct((S, CW), jnp.float32),
        scratch_types=[
            pltpu.VMEM((_GCH,), jnp.int32),
            pltpu.VMEM((_GCH,), jnp.int32),
            pltpu.VMEM((_GCH, CW), jnp.float32),
            pltpu.VMEM((_GCH, CW), jnp.float32),
            pltpu.SemaphoreType.DMA,
            pltpu.SemaphoreType.DMA,
        ],
    )
    def gk(fidx_h, ptab_h, pout_h, idx0, idx1, pb0, pb1, sp0, sp1):
        wid = lax.axis_index("s") * NC + lax.axis_index("c")
        base_w = wid * rows_w

        def start(b, idx_v, pb, sem):
            pltpu.sync_copy(fidx_h.at[pl.ds(b, _GCH)], idx_v)
            return pltpu.async_copy(ptab_h.at[idx_v], pb, sem)

        def drain(b, pb, cp):
            cp.wait()
            pltpu.sync_copy(pb, pout_h.at[pl.ds(b, _GCH)])

        def outer(g, _):
            b0 = base_w + g * (2 * _GCH)
            b1 = b0 + _GCH
            c0 = start(b0, idx0, pb0, sp0)
            c1 = start(b1, idx1, pb1, sp1)
            drain(b0, pb0, c0)
            drain(b1, pb1, c1)
            return 0

        lax.fori_loop(0, nch // 2, outer, 0)

    return gk(fidx, ptab)


def _ball_idx(xyz, new_xyz):
    a2 = (new_xyz ** 2).sum(-1)[:, :, None]
    b2 = (xyz ** 2).sum(-1)[:, None, :]
    ab = jnp.einsum('bik,bjk->bij', new_xyz, xyz)
    d = jnp.sqrt(jnp.maximum(a2 + b2 - 2.0 * ab, 0.0))
    B, M, N = d.shape
    dc = d.reshape(B, M, N // 128, 128)
    nv1, li1 = jax.lax.top_k(-dc, N_SMP)
    gi1 = li1 + (jnp.arange(N // 128, dtype=li1.dtype) * 128)[None, None, :, None]
    negv, c2 = jax.lax.top_k(nv1.reshape(B, M, -1), N_SMP)
    idx = jnp.take_along_axis(gi1.reshape(B, M, -1), c2, axis=2)
    gd = -negv
    cen = jnp.broadcast_to(idx[:, :, :1], idx.shape)
    return jnp.where(gd > RAD, cen, idx).astype(jnp.int32)


def _k1_body(tg_ref, nx_ref, w_ref, wx_ref, y_ref, st_ref):
    i = pl.program_id(0)
    y = jnp.dot(tg_ref[...], w_ref[...], preferred_element_type=jnp.float32)
    t128 = jax.lax.dot_general(
        nx_ref[...], wx_ref[...], (((0,), (0,)), ((), ())),
        preferred_element_type=jnp.float32)
    g_of_s = jax.lax.broadcasted_iota(jnp.int32, (TS, 128), 0) // N_SMP \
        + (i % 8) * 16
    e = (g_of_s == jax.lax.broadcasted_iota(jnp.int32, (TS, 128), 1)
         ).astype(jnp.float32)
    y -= jnp.dot(e, t128, preferred_element_type=jnp.float32)
    y_ref[...] = y
    s = jnp.sum(y, axis=0, keepdims=True)
    q = jnp.sum(y * y, axis=0, keepdims=True)
    part = jnp.concatenate([s, q, jnp.zeros((6, y.shape[1]), jnp.float32)], axis=0)

    @pl.when(i == 0)
    def _():
        st_ref[...] = jnp.zeros_like(st_ref)

    st_ref[...] += part


def _k2_body(y_ref, pr_ref, w_ref, o_ref, st_ref):
    i = pl.program_id(0)
    scale = pr_ref[0:1, :]
    shift = pr_ref[1:2, :]
    z = jnp.maximum(y_ref[...] * scale + shift, 0.0)
    y = jnp.dot(z, w_ref[...], preferred_element_type=jnp.float32)
    o_ref[...] = y
    s = jnp.sum(y, axis=0, keepdims=True)
    q = jnp.sum(y * y, axis=0, keepdims=True)
    part = jnp.concatenate([s, q, jnp.zeros((6, y.shape[1]), jnp.float32)], axis=0)

    @pl.when(i == 0)
    def _():
        st_ref[...] = jnp.zeros_like(st_ref)

    st_ref[...] += part


def _k4_body(y_ref, pr_ref, o_ref):
    scale = pr_ref[0:1, :]
    shift = pr_ref[1:2, :]
    z = jnp.maximum(y_ref[...] * scale + shift, 0.0)
    rows = [jnp.max(z[g * N_SMP:(g + 1) * N_SMP], axis=0, keepdims=True)
            for g in range(TS // N_SMP)]
    o_ref[...] = jnp.concatenate(rows, axis=0)


def _layer1(tg, nx8, w0ext, w0x8, S):
    grid = S // TS
    return pl.pallas_call(
        _k1_body,
        grid=(grid,),
        in_specs=[
            pl.BlockSpec((TS, 256), lambda i: (i, 0)),
            pl.BlockSpec((8, 128), lambda i: (0, i // 8)),
            pl.BlockSpec((256, 128), lambda i: (0, 0)),
            pl.BlockSpec((8, 128), lambda i: (0, 0)),
        ],
        out_specs=[
            pl.BlockSpec((TS, 128), lambda i: (i, 0)),
            pl.BlockSpec((8, 128), lambda i: (0, 0)),
        ],
        out_shape=[
            jax.ShapeDtypeStruct((S, 128), jnp.float32),
            jax.ShapeDtypeStruct((8, 128), jnp.float32),
        ],
    )(tg, nx8, w0ext, w0x8)


def _layer_mid(y, params, wT, S, cout):
    grid = S // TS
    return pl.pallas_call(
        _k2_body,
        grid=(grid,),
        in_specs=[
            pl.BlockSpec((TS, 128), lambda i: (i, 0)),
            pl.BlockSpec((8, 128), lambda i: (0, 0)),
            pl.BlockSpec((128, cout), lambda i: (0, 0)),
        ],
        out_specs=[
            pl.BlockSpec((TS, cout), lambda i: (i, 0)),
            pl.BlockSpec((8, cout), lambda i: (0, 0)),
        ],
        out_shape=[
            jax.ShapeDtypeStruct((S, cout), jnp.float32),
            jax.ShapeDtypeStruct((8, cout), jnp.float32),
        ],
    )(y, params, wT)


def _layer_pool(y, params, S, cout):
    grid = S // TS
    return pl.pallas_call(
        _k4_body,
        grid=(grid,),
        in_specs=[
            pl.BlockSpec((TS, cout), lambda i: (i, 0)),
            pl.BlockSpec((8, cout), lambda i: (0, 0)),
        ],
        out_specs=pl.BlockSpec((TS // N_SMP, cout), lambda i: (i, 0)),
        out_shape=jax.ShapeDtypeStruct((S // N_SMP, cout), jnp.float32),
    )(y, params)


def _mkparams(st, b, gamma, beta, S):
    mean = st[0] / S + b
    var = st[1] / S - (st[0] / S) ** 2
    scale = gamma * jax.lax.rsqrt(var + EPSV)
    shift = beta - mean * scale
    pr = jnp.zeros((8, scale.shape[0]), jnp.float32)
    pr = pr.at[0].set(scale).at[1].set(shift)
    return pr


def kernel(xyz, points, W0, b0, gamma0, beta0, W1, b1, gamma1, beta1,
           W2, b2, gamma2, beta2):
    B, N, _ = xyz.shape
    S = B * N_PT * N_SMP

    cents, new_xyz, nxT = _fps_pallas(xyz)
    idx = _ball_idx(xyz, new_xyz)

    fidx = (idx + (jnp.arange(B, dtype=jnp.int32) * N)[:, None, None]
            ).reshape(S).astype(jnp.int32)
    ptab = jnp.concatenate(
        [points.reshape(B * N, 128), xyz.reshape(B * N, 3),
         jnp.zeros((B * N, 125), jnp.float32)], axis=1)
    tg = _sc_gather(fidx, ptab, S)

    nx8 = jnp.zeros((8, B * N_PT), jnp.float32).at[:3].set(nxT)
    w0x8 = jnp.zeros((8, 128), jnp.float32).at[:3].set(W0[:, :3].T)
    w0ext = jnp.zeros((256, 128), jnp.float32)
    w0ext = w0ext.at[:128].set(W0[:, 3:].T).at[128:131].set(W0[:, :3].T)

    y1, st1 = _layer1(tg, nx8, w0ext, w0x8, S)
    pr1 = _mkparams(st1, b0, gamma0, beta0, S)
    y2, st2 = _layer_mid(y1, pr1, W1.T, S, 128)
    pr2 = _mkparams(st2, b1, gamma1, beta1, S)
    y3, st3 = _layer_mid(y2, pr2, W2.T, S, 256)
    pr3 = _mkparams(st3, b2, gamma2, beta2, S)
    feats = _layer_pool(y3, pr3, S, 256)

    return new_xyz, feats.reshape(B, N_PT, 256)


# speedup vs baseline: 6.5504x; 6.5504x over previous
import functools
import jax
import jax.numpy as jnp
import numpy as np
from jax import lax
from jax.experimental import pallas as pl
from jax.experimental.pallas import tpu as pltpu
from jax.experimental.pallas import tpu_sc as plsc

N_PT = 512
RAD = 0.2
N_SMP = 32
EPSV = 1e-5
TS = 512


def _fps_body(xs_ref, ys_ref, zs_ref, far0_ref, ce_ref, cx_ref, cy_ref, cz_ref):
    B, N = xs_ref.shape
    xs, ys, zs = xs_ref[...], ys_ref[...], zs_ref[...]
    lane = jax.lax.broadcasted_iota(jnp.int32, (B, N), 1)
    slot = jax.lax.broadcasted_iota(jnp.int32, (B, N_PT), 1)

    def body(i, st):
        d, far, ce, cx, cy, cz = st
        mask = lane == far
        cxi = jnp.sum(jnp.where(mask, xs, 0.0), axis=1, keepdims=True)
        cyi = jnp.sum(jnp.where(mask, ys, 0.0), axis=1, keepdims=True)
        czi = jnp.sum(jnp.where(mask, zs, 0.0), axis=1, keepdims=True)
        dx = xs - cxi
        dy = ys - cyi
        dz = zs - czi
        dd = dx * dx + dy * dy + dz * dz
        d = jnp.minimum(d, dd)
        m = jnp.max(d, axis=1, keepdims=True)
        nfar = jnp.min(jnp.where(d == m, lane, N), axis=1, keepdims=True)
        at = slot == i
        ce = jnp.where(at, far, ce)
        cx = jnp.where(at, cxi, cx)
        cy = jnp.where(at, cyi, cy)
        cz = jnp.where(at, czi, cz)
        return (d, nfar, ce, cx, cy, cz)

    d0 = jnp.full((B, N), 1e10, jnp.float32)
    far0 = far0_ref[...][:, :1]
    z512i = jnp.zeros((B, N_PT), jnp.int32)
    z512f = jnp.zeros((B, N_PT), jnp.float32)
    _, _, ce, cx, cy, cz = jax.lax.fori_loop(
        0, N_PT, body, (d0, far0, z512i, z512f, z512f, z512f))
    ce_ref[...] = ce
    cx_ref[...] = cx
    cy_ref[...] = cy
    cz_ref[...] = cz


def _fps_pallas(xyz):
    B, N, _ = xyz.shape
    far0 = jax.random.randint(jax.random.key(42), (B,), 0, N).astype(jnp.int32)
    far0 = jnp.broadcast_to(far0[:, None], (B, 128))
    xT = xyz.transpose(2, 0, 1)
    ce, cx, cy, cz = pl.pallas_call(
        _fps_body,
        out_shape=[
            jax.ShapeDtypeStruct((B, N_PT), jnp.int32),
            jax.ShapeDtypeStruct((B, N_PT), jnp.float32),
            jax.ShapeDtypeStruct((B, N_PT), jnp.float32),
            jax.ShapeDtypeStruct((B, N_PT), jnp.float32),
        ],
    )(xT[0], xT[1], xT[2], far0)
    new_xyz = jnp.stack([cx, cy, cz], axis=-1)
    nxT = jnp.stack([cx, cy, cz], axis=0).reshape(3, B * N_PT)
    return ce, new_xyz, nxT


def _kd_body(nx_ref, px_ref, a2_ref, b2_ref, o_ref):
    c = nx_ref[...]
    p = px_ref[...]
    a2 = a2_ref[...][:, :1]
    b2 = b2_ref[...][0:1, :]
    ab = jax.lax.dot_general(c, p, (((0,), (0,)), ((), ())),
                             preferred_element_type=jnp.float32)
    d = jnp.sqrt(jnp.maximum(a2 + b2 - 2.0 * ab, 0.0))
    bits = jax.lax.bitcast_convert_type(d, jnp.int32)

    def bis(_, st):
        lo, hi = st
        mid = jax.lax.shift_right_arithmetic(lo + hi, 1)
        cnt = jnp.sum((bits <= mid).astype(jnp.int32), axis=1, keepdims=True)
        ge = cnt >= N_SMP
        return (jnp.where(ge, lo, mid), jnp.where(ge, mid, hi))

    lo0 = jnp.zeros((N_PT, 1), jnp.int32)
    hi0 = jnp.full((N_PT, 1), 0x40000000, jnp.int32)
    _, hi = jax.lax.fori_loop(0, 30, bis, (lo0, hi0))
    t32 = jax.lax.bitcast_convert_type(hi, jnp.float32)
    thr = jnp.minimum(t32, RAD)
    selm = d <= thr
    sel = jnp.where(selm, 1.0, 0.0)
    rowi = jax.lax.broadcasted_iota(jnp.int32, (2048, 128), 0)
    coli = jax.lax.broadcasted_iota(jnp.int32, (2048, 128), 1)
    pf = [jnp.dot(sel, jnp.where(rowi <= kb * 128 + coli, 1.0, 0.0),
                  preferred_element_type=jnp.float32) for kb in range(16)]
    pf = jnp.concatenate(pf, axis=1).astype(jnp.int32)
    pc = jnp.where(selm & (pf <= N_SMP), pf, 0)
    gmin = jnp.min(d, axis=1, keepdims=True)
    lane = jax.lax.broadcasted_iota(jnp.int32, d.shape, 1)
    cen = jnp.min(jnp.where(d == gmin, lane, d.shape[1]), axis=1,
                  keepdims=True)
    cols = []
    for i in range(1, N_SMP + 1):
        gi = jnp.max(jnp.where(pc == i, lane, -1), axis=1, keepdims=True)
        cols.append(jnp.where(gi < 0, cen, gi))
    cols.append(jnp.zeros((N_PT, 128 - N_SMP), jnp.int32))
    o_ref[...] = jnp.concatenate(cols, axis=1)


def _kd(nx8, xyz8, a2c, b2r, B):
    return pl.pallas_call(
        _kd_body,
        grid=(B,),
        in_specs=[
            pl.BlockSpec((8, N_PT), lambda b: (0, b)),
            pl.BlockSpec((8, 2048), lambda b: (0, b)),
            pl.BlockSpec((N_PT, 128), lambda b: (b, 0)),
            pl.BlockSpec((8, 2048), lambda b: (0, b)),
        ],
        out_specs=pl.BlockSpec((N_PT, 128), lambda b: (b, 0)),
        out_shape=jax.ShapeDtypeStruct((B * N_PT, 128), jnp.int32),
    )(nx8, xyz8, a2c, b2r)


_GCH = 128


def _sc_gather(fidx, ptab, S):
    mesh = plsc.VectorSubcoreMesh(core_axis_name="c", subcore_axis_name="s")
    NC, NS = 2, 16
    NW = NC * NS
    rows_w = S // NW
    nch = rows_w // _GCH
    CW = ptab.shape[1]

    @functools.partial(
        pl.kernel,
        mesh=mesh,
        out_type=jax.ShapeDtypeStruct((S, CW), jnp.float32),
        scratch_types=[
            pltpu.VMEM((_GCH,), jnp.int32),
            pltpu.VMEM((_GCH,), jnp.int32),
            pltpu.VMEM((_GCH, CW), jnp.float32),
            pltpu.VMEM((_GCH, CW), jnp.float32),
            pltpu.SemaphoreType.DMA,
            pltpu.SemaphoreType.DMA,
        ],
    )
    def gk(fidx_h, ptab_h, pout_h, idx0, idx1, pb0, pb1, sp0, sp1):
        wid = lax.axis_index("s") * NC + lax.axis_index("c")
        base_w = wid * rows_w

        def start(b, idx_v, pb, sem):
            pltpu.sync_copy(fidx_h.at[pl.ds(b, _GCH)], idx_v)
            return pltpu.async_copy(ptab_h.at[idx_v], pb, sem)

        def drain(b, pb, cp):
            cp.wait()
            pltpu.sync_copy(pb, pout_h.at[pl.ds(b, _GCH)])

        def outer(g, _):
            b0 = base_w + g * (2 * _GCH)
            b1 = b0 + _GCH
            c0 = start(b0, idx0, pb0, sp0)
            c1 = start(b1, idx1, pb1, sp1)
            drain(b0, pb0, c0)
            drain(b1, pb1, c1)
            return 0

        lax.fori_loop(0, nch // 2, outer, 0)

    return gk(fidx, ptab)


def _k1_body(tg_ref, nx_ref, w_ref, wx_ref, y_ref, st_ref):
    i = pl.program_id(0)
    y = jnp.dot(tg_ref[...], w_ref[...], preferred_element_type=jnp.float32)
    t128 = jax.lax.dot_general(
        nx_ref[...], wx_ref[...], (((0,), (0,)), ((), ())),
        preferred_element_type=jnp.float32)
    g_of_s = jax.lax.broadcasted_iota(jnp.int32, (TS, 128), 0) // N_SMP \
        + (i % 8) * 16
    e = (g_of_s == jax.lax.broadcasted_iota(jnp.int32, (TS, 128), 1)
         ).astype(jnp.float32)
    y -= jnp.dot(e, t128, preferred_element_type=jnp.float32)
    y_ref[...] = y
    s = jnp.sum(y, axis=0, keepdims=True)
    q = jnp.sum(y * y, axis=0, keepdims=True)
    part = jnp.concatenate([s, q, jnp.zeros((6, y.shape[1]), jnp.float32)], axis=0)

    @pl.when(i == 0)
    def _():
        st_ref[...] = jnp.zeros_like(st_ref)

    st_ref[...] += part


def _k2_body(y_ref, pr_ref, w_ref, o_ref, st_ref):
    i = pl.program_id(0)
    scale = pr_ref[0:1, :]
    shift = pr_ref[1:2, :]
    z = jnp.maximum(y_ref[...] * scale + shift, 0.0)
    y = jnp.dot(z, w_ref[...], preferred_element_type=jnp.float32)
    o_ref[...] = y
    s = jnp.sum(y, axis=0, keepdims=True)
    q = jnp.sum(y * y, axis=0, keepdims=True)
    part = jnp.concatenate([s, q, jnp.zeros((6, y.shape[1]), jnp.float32)], axis=0)

    @pl.when(i == 0)
    def _():
        st_ref[...] = jnp.zeros_like(st_ref)

    st_ref[...] += part


def _k4_body(y_ref, pr_ref, o_ref):
    scale = pr_ref[0:1, :]
    shift = pr_ref[1:2, :]
    z = jnp.maximum(y_ref[...] * scale + shift, 0.0)
    rows = [jnp.max(z[g * N_SMP:(g + 1) * N_SMP], axis=0, keepdims=True)
            for g in range(TS // N_SMP)]
    o_ref[...] = jnp.concatenate(rows, axis=0)


def _layer1(tg, nx8, w0ext, w0x8, S):
    grid = S // TS
    return pl.pallas_call(
        _k1_body,
        grid=(grid,),
        in_specs=[
            pl.BlockSpec((TS, 256), lambda i: (i, 0)),
            pl.BlockSpec((8, 128), lambda i: (0, i // 8)),
            pl.BlockSpec((256, 128), lambda i: (0, 0)),
            pl.BlockSpec((8, 128), lambda i: (0, 0)),
        ],
        out_specs=[
            pl.BlockSpec((TS, 128), lambda i: (i, 0)),
            pl.BlockSpec((8, 128), lambda i: (0, 0)),
        ],
        out_shape=[
            jax.ShapeDtypeStruct((S, 128), jnp.float32),
            jax.ShapeDtypeStruct((8, 128), jnp.float32),
        ],
    )(tg, nx8, w0ext, w0x8)


def _layer_mid(y, params, wT, S, cout):
    grid = S // TS
    return pl.pallas_call(
        _k2_body,
        grid=(grid,),
        in_specs=[
            pl.BlockSpec((TS, 128), lambda i: (i, 0)),
            pl.BlockSpec((8, 128), lambda i: (0, 0)),
            pl.BlockSpec((128, cout), lambda i: (0, 0)),
        ],
        out_specs=[
            pl.BlockSpec((TS, cout), lambda i: (i, 0)),
            pl.BlockSpec((8, cout), lambda i: (0, 0)),
        ],
        out_shape=[
            jax.ShapeDtypeStruct((S, cout), jnp.float32),
            jax.ShapeDtypeStruct((8, cout), jnp.float32),
        ],
    )(y, params, wT)


def _layer_pool(y, params, S, cout):
    grid = S // TS
    return pl.pallas_call(
        _k4_body,
        grid=(grid,),
        in_specs=[
            pl.BlockSpec((TS, cout), lambda i: (i, 0)),
            pl.BlockSpec((8, cout), lambda i: (0, 0)),
        ],
        out_specs=pl.BlockSpec((TS // N_SMP, cout), lambda i: (i, 0)),
        out_shape=jax.ShapeDtypeStruct((S // N_SMP, cout), jnp.float32),
    )(y, params)


def _mkparams(st, b, gamma, beta, S):
    mean = st[0] / S + b
    var = st[1] / S - (st[0] / S) ** 2
    scale = gamma * jax.lax.rsqrt(var + EPSV)
    shift = beta - mean * scale
    pr = jnp.zeros((8, scale.shape[0]), jnp.float32)
    pr = pr.at[0].set(scale).at[1].set(shift)
    return pr


def kernel(xyz, points, W0, b0, gamma0, beta0, W1, b1, gamma1, beta1,
           W2, b2, gamma2, beta2):
    B, N, _ = xyz.shape
    S = B * N_PT * N_SMP

    cents, new_xyz, nxT = _fps_pallas(xyz)

    xyzT3 = xyz.transpose(2, 0, 1).reshape(3, B * N)
    nx8 = jnp.zeros((8, B * N_PT), jnp.float32).at[:3].set(nxT)
    xyz8 = jnp.zeros((8, B * N), jnp.float32).at[:3].set(xyzT3)
    a2v = (new_xyz ** 2).sum(-1).reshape(B * N_PT)
    a2c = jnp.broadcast_to(a2v[:, None], (B * N_PT, 128))
    b2v = (xyz ** 2).sum(-1).reshape(B * N)
    b2r = jnp.broadcast_to(b2v[None, :], (8, B * N))
    idxmat = _kd(nx8, xyz8, a2c, b2r, B)

    offs = (jnp.arange(B, dtype=jnp.int32) * N).repeat(N_PT)[:, None]
    fidx = (idxmat[:, :N_SMP] + offs).reshape(S)
    ptab = jnp.concatenate(
        [points.reshape(B * N, 128), xyz.reshape(B * N, 3),
         jnp.zeros((B * N, 125), jnp.float32)], axis=1)
    tg = _sc_gather(fidx, ptab, S)

    w0x8 = jnp.zeros((8, 128), jnp.float32).at[:3].set(W0[:, :3].T)
    w0ext = jnp.zeros((256, 128), jnp.float32)
    w0ext = w0ext.at[:128].set(W0[:, 3:].T).at[128:131].set(W0[:, :3].T)

    y1, st1 = _layer1(tg, nx8, w0ext, w0x8, S)
    pr1 = _mkparams(st1, b0, gamma0, beta0, S)
    y2, st2 = _layer_mid(y1, pr1, W1.T, S, 128)
    pr2 = _mkparams(st2, b1, gamma1, beta1, S)
    y3, st3 = _layer_mid(y2, pr2, W2.T, S, 256)
    pr3 = _mkparams(st3, b2, gamma2, beta2, S)
    feats = _layer_pool(y3, pr3, S, 256)

    return new_xyz, feats.reshape(B, N_PT, 256)


# speedup vs baseline: 6.7811x; 1.0352x over previous
import functools
import jax
import jax.numpy as jnp
import numpy as np
from jax import lax
from jax.experimental import pallas as pl
from jax.experimental.pallas import tpu as pltpu
from jax.experimental.pallas import tpu_sc as plsc

N_PT = 512
RAD = 0.2
N_SMP = 32
EPSV = 1e-5
TS = 512


def _fps_body(xs_ref, ys_ref, zs_ref, far0_ref, ce_ref, cx_ref, cy_ref, cz_ref):
    B, N = xs_ref.shape
    xs, ys, zs = xs_ref[...], ys_ref[...], zs_ref[...]
    lane = jax.lax.broadcasted_iota(jnp.int32, (B, N), 1)
    slot = jax.lax.broadcasted_iota(jnp.int32, (B, N_PT), 1)

    def body(i, st):
        d, far, ce, cx, cy, cz = st
        mask = lane == far
        cxi = jnp.sum(jnp.where(mask, xs, 0.0), axis=1, keepdims=True)
        cyi = jnp.sum(jnp.where(mask, ys, 0.0), axis=1, keepdims=True)
        czi = jnp.sum(jnp.where(mask, zs, 0.0), axis=1, keepdims=True)
        dx = xs - cxi
        dy = ys - cyi
        dz = zs - czi
        dd = dx * dx + dy * dy + dz * dz
        d = jnp.minimum(d, dd)
        m = jnp.max(d, axis=1, keepdims=True)
        nfar = jnp.min(jnp.where(d == m, lane, N), axis=1, keepdims=True)
        at = slot == i
        ce = jnp.where(at, far, ce)
        cx = jnp.where(at, cxi, cx)
        cy = jnp.where(at, cyi, cy)
        cz = jnp.where(at, czi, cz)
        return (d, nfar, ce, cx, cy, cz)

    d0 = jnp.full((B, N), 1e10, jnp.float32)
    far0 = far0_ref[...][:, :1]
    z512i = jnp.zeros((B, N_PT), jnp.int32)
    z512f = jnp.zeros((B, N_PT), jnp.float32)
    _, _, ce, cx, cy, cz = jax.lax.fori_loop(
        0, N_PT, body, (d0, far0, z512i, z512f, z512f, z512f))
    ce_ref[...] = ce
    cx_ref[...] = cx
    cy_ref[...] = cy
    cz_ref[...] = cz


def _fps_pallas(xyz):
    B, N, _ = xyz.shape
    far0 = jax.random.randint(jax.random.key(42), (B,), 0, N).astype(jnp.int32)
    far0 = jnp.broadcast_to(far0[:, None], (B, 128))
    xT = xyz.transpose(2, 0, 1)
    ce, cx, cy, cz = pl.pallas_call(
        _fps_body,
        out_shape=[
            jax.ShapeDtypeStruct((B, N_PT), jnp.int32),
            jax.ShapeDtypeStruct((B, N_PT), jnp.float32),
            jax.ShapeDtypeStruct((B, N_PT), jnp.float32),
            jax.ShapeDtypeStruct((B, N_PT), jnp.float32),
        ],
    )(xT[0], xT[1], xT[2], far0)
    new_xyz = jnp.stack([cx, cy, cz], axis=-1)
    nxT = jnp.stack([cx, cy, cz], axis=0).reshape(3, B * N_PT)
    return ce, new_xyz, nxT


def _kd_body(nx_ref, px_ref, a2_ref, b2_ref, o_ref):
    c = nx_ref[...]
    p = px_ref[...]
    a2 = a2_ref[...][:, :1]
    b2 = b2_ref[...][0:1, :]
    ab = jax.lax.dot_general(c, p, (((0,), (0,)), ((), ())),
                             preferred_element_type=jnp.float32)
    d = jnp.sqrt(jnp.maximum(a2 + b2 - 2.0 * ab, 0.0))
    bits = jax.lax.bitcast_convert_type(d, jnp.int32)

    def bis(_, st):
        lo, hi = st
        mid = jax.lax.shift_right_arithmetic(lo + hi, 1)
        cnt = jnp.sum((bits <= mid).astype(jnp.int32), axis=1, keepdims=True)
        ge = cnt >= N_SMP
        return (jnp.where(ge, lo, mid), jnp.where(ge, mid, hi))

    lo0 = jnp.zeros((N_PT, 1), jnp.int32)
    hi0 = jnp.full((N_PT, 1), 0x40000000, jnp.int32)
    _, hi = jax.lax.fori_loop(0, 30, bis, (lo0, hi0))
    t32 = jax.lax.bitcast_convert_type(hi, jnp.float32)
    thr = jnp.minimum(t32, RAD)
    selm = d <= thr
    sel = jnp.where(selm, 1.0, 0.0)
    rowi = jax.lax.broadcasted_iota(jnp.int32, (2048, 128), 0)
    coli = jax.lax.broadcasted_iota(jnp.int32, (2048, 128), 1)
    pf = [jnp.dot(sel, jnp.where(rowi <= kb * 128 + coli, 1.0, 0.0),
                  preferred_element_type=jnp.float32) for kb in range(16)]
    pf = jnp.concatenate(pf, axis=1).astype(jnp.int32)
    pc = jnp.where(selm & (pf <= N_SMP), pf, 0)
    gmin = jnp.min(d, axis=1, keepdims=True)
    lane = jax.lax.broadcasted_iota(jnp.int32, d.shape, 1)
    cen = jnp.min(jnp.where(d == gmin, lane, d.shape[1]), axis=1,
                  keepdims=True)
    cols = []
    for i in range(1, N_SMP + 1):
        gi = jnp.max(jnp.where(pc == i, lane, -1), axis=1, keepdims=True)
        cols.append(jnp.where(gi < 0, cen, gi))
    cols.append(jnp.zeros((N_PT, 128 - N_SMP), jnp.int32))
    o_ref[...] = jnp.concatenate(cols, axis=1)


def _kd(nx8, xyz8, a2c, b2r, B):
    return pl.pallas_call(
        _kd_body,
        grid=(B,),
        in_specs=[
            pl.BlockSpec((8, N_PT), lambda b: (0, b)),
            pl.BlockSpec((8, 2048), lambda b: (0, b)),
            pl.BlockSpec((N_PT, 128), lambda b: (b, 0)),
            pl.BlockSpec((8, 2048), lambda b: (0, b)),
        ],
        out_specs=pl.BlockSpec((N_PT, 128), lambda b: (b, 0)),
        out_shape=jax.ShapeDtypeStruct((B * N_PT, 128), jnp.int32),
    )(nx8, xyz8, a2c, b2r)


_GCH = 128


def _sc_gather(fidx, ptab, S):
    mesh = plsc.VectorSubcoreMesh(core_axis_name="c", subcore_axis_name="s")
    NC, NS = 2, 16
    NW = NC * NS
    rows_w = S // NW
    nch = rows_w // _GCH
    CW = ptab.shape[1]

    @functools.partial(
        pl.kernel,
        mesh=mesh,
        out_type=jax.ShapeDtypeStruct((S, CW), jnp.float32),
        scratch_types=[
            pltpu.VMEM((_GCH,), jnp.int32),
            pltpu.VMEM((_GCH,), jnp.int32),
            pltpu.VMEM((_GCH, CW), jnp.float32),
            pltpu.VMEM((_GCH, CW), jnp.float32),
            pltpu.SemaphoreType.DMA,
            pltpu.SemaphoreType.DMA,
        ],
    )
    def gk(fidx_h, ptab_h, pout_h, idx0, idx1, pb0, pb1, sp0, sp1):
        wid = lax.axis_index("s") * NC + lax.axis_index("c")
        base_w = wid * rows_w

        def start(b, idx_v, pb, sem):
            pltpu.sync_copy(fidx_h.at[pl.ds(b, _GCH)], idx_v)
            return pltpu.async_copy(ptab_h.at[idx_v], pb, sem)

        def drain(b, pb, cp):
            cp.wait()
            pltpu.sync_copy(pb, pout_h.at[pl.ds(b, _GCH)])

        def outer(g, _):
            b0 = base_w + g * (2 * _GCH)
            b1 = b0 + _GCH
            c0 = start(b0, idx0, pb0, sp0)
            c1 = start(b1, idx1, pb1, sp1)
            drain(b0, pb0, c0)
            drain(b1, pb1, c1)
            return 0

        lax.fori_loop(0, nch // 2, outer, 0)

    return gk(fidx, ptab)


def _k1_body(tg_ref, nx_ref, w_ref, wx_ref, y_ref, st_ref):
    i = pl.program_id(0)
    y = jnp.dot(tg_ref[...], w_ref[...], preferred_element_type=jnp.float32)
    t128 = jax.lax.dot_general(
        nx_ref[...], wx_ref[...], (((0,), (0,)), ((), ())),
        preferred_element_type=jnp.float32)
    g_of_s = jax.lax.broadcasted_iota(jnp.int32, (TS, 128), 0) // N_SMP \
        + (i % 8) * 16
    e = (g_of_s == jax.lax.broadcasted_iota(jnp.int32, (TS, 128), 1)
         ).astype(jnp.float32)
    y -= jnp.dot(e, t128, preferred_element_type=jnp.float32)
    y_ref[...] = y.astype(jnp.bfloat16)
    s = jnp.sum(y, axis=0, keepdims=True)
    q = jnp.sum(y * y, axis=0, keepdims=True)
    part = jnp.concatenate([s, q, jnp.zeros((6, y.shape[1]), jnp.float32)], axis=0)

    @pl.when(i == 0)
    def _():
        st_ref[...] = jnp.zeros_like(st_ref)

    st_ref[...] += part


def _k2_body(y_ref, pr_ref, w_ref, o_ref, st_ref):
    i = pl.program_id(0)
    scale = pr_ref[0:1, :]
    shift = pr_ref[1:2, :]
    z = jnp.maximum(y_ref[...].astype(jnp.float32) * scale + shift, 0.0)
    y = jnp.dot(z, w_ref[...], preferred_element_type=jnp.float32)
    o_ref[...] = y.astype(jnp.bfloat16)
    s = jnp.sum(y, axis=0, keepdims=True)
    q = jnp.sum(y * y, axis=0, keepdims=True)
    part = jnp.concatenate([s, q, jnp.zeros((6, y.shape[1]), jnp.float32)], axis=0)

    @pl.when(i == 0)
    def _():
        st_ref[...] = jnp.zeros_like(st_ref)

    st_ref[...] += part


def _k4_body(y_ref, pr_ref, o_ref):
    scale = pr_ref[0:1, :]
    shift = pr_ref[1:2, :]
    z = jnp.maximum(y_ref[...].astype(jnp.float32) * scale + shift, 0.0)
    rows = [jnp.max(z[g * N_SMP:(g + 1) * N_SMP], axis=0, keepdims=True)
            for g in range(TS // N_SMP)]
    o_ref[...] = jnp.concatenate(rows, axis=0)


def _layer1(tg, nx8, w0ext, w0x8, S):
    grid = S // TS
    return pl.pallas_call(
        _k1_body,
        grid=(grid,),
        in_specs=[
            pl.BlockSpec((TS, 256), lambda i: (i, 0)),
            pl.BlockSpec((8, 128), lambda i: (0, i // 8)),
            pl.BlockSpec((256, 128), lambda i: (0, 0)),
            pl.BlockSpec((8, 128), lambda i: (0, 0)),
        ],
        out_specs=[
            pl.BlockSpec((TS, 128), lambda i: (i, 0)),
            pl.BlockSpec((8, 128), lambda i: (0, 0)),
        ],
        out_shape=[
            jax.ShapeDtypeStruct((S, 128), jnp.bfloat16),
            jax.ShapeDtypeStruct((8, 128), jnp.float32),
        ],
    )(tg, nx8, w0ext, w0x8)


def _layer_mid(y, params, wT, S, cout):
    grid = S // TS
    return pl.pallas_call(
        _k2_body,
        grid=(grid,),
        in_specs=[
            pl.BlockSpec((TS, 128), lambda i: (i, 0)),
            pl.BlockSpec((8, 128), lambda i: (0, 0)),
            pl.BlockSpec((128, cout), lambda i: (0, 0)),
        ],
        out_specs=[
            pl.BlockSpec((TS, cout), lambda i: (i, 0)),
            pl.BlockSpec((8, cout), lambda i: (0, 0)),
        ],
        out_shape=[
            jax.ShapeDtypeStruct((S, cout), jnp.bfloat16),
            jax.ShapeDtypeStruct((8, cout), jnp.float32),
        ],
    )(y, params, wT)


def _layer_pool(y, params, S, cout):
    grid = S // TS
    return pl.pallas_call(
        _k4_body,
        grid=(grid,),
        in_specs=[
            pl.BlockSpec((TS, cout), lambda i: (i, 0)),
            pl.BlockSpec((8, cout), lambda i: (0, 0)),
        ],
        out_specs=pl.BlockSpec((TS // N_SMP, cout), lambda i: (i, 0)),
        out_shape=jax.ShapeDtypeStruct((S // N_SMP, cout), jnp.float32),
    )(y, params)


def _mkparams(st, b, gamma, beta, S):
    mean = st[0] / S + b
    var = st[1] / S - (st[0] / S) ** 2
    scale = gamma * jax.lax.rsqrt(var + EPSV)
    shift = beta - mean * scale
    pr = jnp.zeros((8, scale.shape[0]), jnp.float32)
    pr = pr.at[0].set(scale).at[1].set(shift)
    return pr


def kernel(xyz, points, W0, b0, gamma0, beta0, W1, b1, gamma1, beta1,
           W2, b2, gamma2, beta2):
    B, N, _ = xyz.shape
    S = B * N_PT * N_SMP

    cents, new_xyz, nxT = _fps_pallas(xyz)

    xyzT3 = xyz.transpose(2, 0, 1).reshape(3, B * N)
    nx8 = jnp.zeros((8, B * N_PT), jnp.float32).at[:3].set(nxT)
    xyz8 = jnp.zeros((8, B * N), jnp.float32).at[:3].set(xyzT3)
    a2v = (new_xyz ** 2).sum(-1).reshape(B * N_PT)
    a2c = jnp.broadcast_to(a2v[:, None], (B * N_PT, 128))
    b2v = (xyz ** 2).sum(-1).reshape(B * N)
    b2r = jnp.broadcast_to(b2v[None, :], (8, B * N))
    idxmat = _kd(nx8, xyz8, a2c, b2r, B)

    offs = (jnp.arange(B, dtype=jnp.int32) * N).repeat(N_PT)[:, None]
    fidx = (idxmat[:, :N_SMP] + offs).reshape(S)
    ptab = jnp.concatenate(
        [points.reshape(B * N, 128), xyz.reshape(B * N, 3),
         jnp.zeros((B * N, 125), jnp.float32)], axis=1)
    tg = _sc_gather(fidx, ptab, S)

    w0x8 = jnp.zeros((8, 128), jnp.float32).at[:3].set(W0[:, :3].T)
    w0ext = jnp.zeros((256, 128), jnp.float32)
    w0ext = w0ext.at[:128].set(W0[:, 3:].T).at[128:131].set(W0[:, :3].T)

    y1, st1 = _layer1(tg, nx8, w0ext, w0x8, S)
    pr1 = _mkparams(st1, b0, gamma0, beta0, S)
    y2, st2 = _layer_mid(y1, pr1, W1.T, S, 128)
    pr2 = _mkparams(st2, b1, gamma1, beta1, S)
    y3, st3 = _layer_mid(y2, pr2, W2.T, S, 256)
    pr3 = _mkparams(st3, b2, gamma2, beta2, S)
    feats = _layer_pool(y3, pr3, S, 256)

    return new_xyz, feats.reshape(B, N_PT, 256)
